# baseline (speedup 1.0000x reference)
import dataclasses
import functools

import jax
from jax import lax
import jax.numpy as jnp
from jax.experimental import pallas as pl
from jax.experimental.pallas import tpu as pltpu
from jax.experimental.pallas import tpu_sc as plsc

SEQ = 4096
BATCH = 4
HIDDEN = 4096
ROWS = SEQ * BATCH
MAXPOS = 8192

NW = 32
BKT = MAXPOS // NW
SB = 8
NS = BKT // SB
NRB = 2
NOUT = 16

_vector_mesh = plsc.VectorSubcoreMesh(
    core_axis_name="core", subcore_axis_name="subcore"
)

_cp = pltpu.CompilerParams()
if "needs_layout_passes" in pltpu.CompilerParams.__dataclass_fields__:
  _cp = dataclasses.replace(_cp, needs_layout_passes=False)


@jax.jit
def _sc_scatter_gather(table, indices):
  @functools.partial(
      pl.kernel,
      out_type=jax.ShapeDtypeStruct((ROWS, HIDDEN), table.dtype),
      mesh=_vector_mesh,
      compiler_params=_cp,
      scratch_types=[
          pltpu.VMEM((ROWS,), jnp.int32),
          pltpu.VMEM((ROWS,), jnp.int32),
          pltpu.VMEM((ROWS,), jnp.int32),
          *[pltpu.VMEM((SB, HIDDEN), table.dtype) for _ in range(NRB)],
          *[pltpu.SemaphoreType.DMA for _ in range(NRB)],
          *[pltpu.SemaphoreType.DMA for _ in range(NRB)],
      ],
  )
  def kern(table_hbm, idx_hbm, out_hbm, idxs, bkt_buf, sub_buf,
           *scratch):
    rowbufs = scratch[:NRB]
    lsems = scratch[NRB:2 * NRB]
    osems = scratch[2 * NRB:]
    wid = lax.axis_index("subcore") * 2 + lax.axis_index("core")
    r0 = wid * BKT

    def start_load(s, b):
      pltpu.async_copy(table_hbm.at[pl.ds(r0 + s * SB, SB)], rowbufs[b],
                       lsems[b])

    def wait_load(b):
      pltpu.make_async_copy(table_hbm.at[pl.ds(0, SB)], rowbufs[b],
                            lsems[b]).wait()

    def wait_one_write(b):
      pltpu.make_async_copy(rowbufs[b].at[pl.ds(0, 1)],
                            out_hbm.at[pl.ds(0, 1)], osems[b]).wait()

    start_load(0, 0)

    pltpu.sync_copy(idx_hbm, idxs)

    iota = lax.iota(jnp.int32, 16)
    ones = jnp.ones((16,), jnp.int32)
    zero16 = jnp.zeros((16,), jnp.int32)

    def p1_body(i, off):
      for u in range(2):
        v = idxs[pl.ds(i * 32 + u * 16, 16)]
        local = v - r0
        m = (local >= 0) & (local < BKT)
        packed = (iota + (i * 32 + u * 16)) * 256 + local
        dst = off + plsc.cumsum(jnp.where(m, ones, zero16)) - 1
        plsc.store_scatter(bkt_buf, [dst], packed, mask=m)
        off = off + plsc.all_reduce_population_count(m)
      return off

    off = lax.fori_loop(0, ROWS // 32, p1_body, zero16)
    total = jnp.max(off)
    nvec = (total + 15) // 16

    def s_pair_body(t, kprev):
      s0 = t * NRB
      for b in range(NRB):
        s = s0 + b
        bn = (b + 1) % NRB

        def p2_body(j, off2, s=s):
          v = bkt_buf[pl.ds(j * 16, 16)]
          local = v & 255
          m = ((local >= s * SB) & (local < (s + 1) * SB)
               & (iota + j * 16 < total))
          packed = (v >> 8) * 8 + (local & 7)
          dst = off2 + plsc.cumsum(jnp.where(m, ones, zero16)) - 1
          plsc.store_scatter(sub_buf, [dst], packed, mask=m)
          return off2 + plsc.all_reduce_population_count(m)

        k2 = jnp.max(lax.fori_loop(0, nvec, p2_body, zero16))

        def late_drain_body(j, carry, bn=bn):
          wait_one_write(bn)
          return carry

        lax.fori_loop(0, jnp.minimum(kprev[bn], NOUT), late_drain_body, 0)
        kprev = tuple(jnp.int32(0) if i == bn else kprev[i]
                      for i in range(NRB))

        @pl.when(s + 1 < NS)
        def _(s=s, bn=bn):
          start_load(s + 1, bn)

        wait_load(b)

        def wr_body(jv, carry, b=b, k2=k2):
          v = sub_buf[pl.ds(jv * 16, 16)]
          full = jv * 16 + 16 <= k2

          @pl.when(full)
          def _(b=b, jv=jv):
            @pl.when(jv >= 1)
            def _(b=b):
              pltpu.make_async_copy(table_hbm.at[pl.ds(0, 16)],
                                    out_hbm.at[pl.ds(0, 16)],
                                    osems[b]).wait()

            for l in range(16):
              w = v[l]
              pltpu.async_copy(rowbufs[b].at[pl.ds(w & 7, 1)],
                               out_hbm.at[pl.ds(w >> 3, 1)], osems[b])

          @pl.when(jnp.logical_not(full))
          def _(b=b, jv=jv):
            for l in range(16):
              j = jv * 16 + l
              valid = j < k2

              @pl.when(valid)
              def _(l=l, b=b):
                w = v[l]
                pltpu.async_copy(rowbufs[b].at[pl.ds(w & 7, 1)],
                                 out_hbm.at[pl.ds(w >> 3, 1)], osems[b])

              @pl.when(valid & (j >= NOUT))
              def _(b=b):
                wait_one_write(b)

          return carry

        lax.fori_loop(0, (k2 + 15) // 16, wr_body, 0)
        kprev = tuple(k2 if i == b else kprev[i] for i in range(NRB))
      return kprev

    kfin = lax.fori_loop(0, NS // NRB, s_pair_body,
                         tuple(jnp.int32(0) for _ in range(NRB)))

    for b in range(NRB):

      def fin_drain_body(j, carry, b=b):
        wait_one_write(b)
        return carry

      lax.fori_loop(0, jnp.minimum(kfin[b], NOUT), fin_drain_body, 0)

  return kern(table, indices)


def kernel(position_ids, embedding_table):
  idx = jnp.transpose(position_ids).reshape(ROWS).astype(jnp.int32)
  out = _sc_scatter_gather(embedding_table, idx)
  return out.reshape(BATCH, SEQ, HIDDEN)


# speedup vs baseline: 1.0076x; 1.0076x over previous
import dataclasses
import functools

import jax
from jax import lax
import jax.numpy as jnp
from jax.experimental import pallas as pl
from jax.experimental.pallas import tpu as pltpu
from jax.experimental.pallas import tpu_sc as plsc

SEQ = 4096
BATCH = 4
HIDDEN = 4096
ROWS = SEQ * BATCH
MAXPOS = 8192

NW = 32
BKT = MAXPOS // NW
SB = 8
NS = BKT // SB
NRB = 2
NOUT = 16

_vector_mesh = plsc.VectorSubcoreMesh(
    core_axis_name="core", subcore_axis_name="subcore"
)

_cp = pltpu.CompilerParams()
if "needs_layout_passes" in pltpu.CompilerParams.__dataclass_fields__:
  _cp = dataclasses.replace(_cp, needs_layout_passes=False)


@jax.jit
def _sc_scatter_gather(table, indices):
  @functools.partial(
      pl.kernel,
      out_type=jax.ShapeDtypeStruct((ROWS, HIDDEN), table.dtype),
      mesh=_vector_mesh,
      compiler_params=_cp,
      scratch_types=[
          pltpu.VMEM((ROWS,), jnp.int32),
          pltpu.VMEM((ROWS,), jnp.int32),
          pltpu.VMEM((ROWS,), jnp.int32),
          *[pltpu.VMEM((SB, HIDDEN), table.dtype) for _ in range(NRB)],
          *[pltpu.SemaphoreType.DMA for _ in range(NRB)],
          *[pltpu.SemaphoreType.DMA for _ in range(NRB)],
      ],
  )
  def kern(table_hbm, idx_hbm, out_hbm, idxs, bkt_buf, sub_buf,
           *scratch):
    rowbufs = scratch[:NRB]
    lsems = scratch[NRB:2 * NRB]
    osems = scratch[2 * NRB:]
    wid = lax.axis_index("subcore") * 2 + lax.axis_index("core")
    r0 = wid * BKT

    def start_load(s, b):
      pltpu.async_copy(table_hbm.at[pl.ds(r0 + s * SB, SB)], rowbufs[b],
                       lsems[b])

    def wait_load(b):
      pltpu.make_async_copy(table_hbm.at[pl.ds(0, SB)], rowbufs[b],
                            lsems[b]).wait()

    def wait_one_write(b):
      pltpu.make_async_copy(rowbufs[b].at[pl.ds(0, 1)],
                            out_hbm.at[pl.ds(0, 1)], osems[b]).wait()

    start_load(0, 0)

    pltpu.sync_copy(idx_hbm, idxs)

    iota = lax.iota(jnp.int32, 16)
    ones = jnp.ones((16,), jnp.int32)
    zero16 = jnp.zeros((16,), jnp.int32)

    def p1_body(i, off):
      for u in range(2):
        v = idxs[pl.ds(i * 32 + u * 16, 16)]
        local = v - r0
        m = (local >= 0) & (local < BKT)
        packed = (iota + (i * 32 + u * 16)) * 256 + local
        dst = off + plsc.cumsum(jnp.where(m, ones, zero16)) - 1
        plsc.store_scatter(bkt_buf, [dst], packed, mask=m)
        off = off + plsc.all_reduce_population_count(m)
      return off

    off = lax.fori_loop(0, ROWS // 32, p1_body, zero16)
    total = jnp.max(off)
    nvec = (total + 15) // 16

    def s_pair_body(t, kprev):
      s0 = t * NRB
      for b in range(NRB):
        s = s0 + b
        bn = (b + 1) % NRB

        def p2_body(j, off2, s=s):
          v = bkt_buf[pl.ds(j * 16, 16)]
          local = v & 255
          m = ((local >= s * SB) & (local < (s + 1) * SB)
               & (iota + j * 16 < total))
          packed = (v >> 8) * 8 + (local & 7)
          dst = off2 + plsc.cumsum(jnp.where(m, ones, zero16)) - 1
          plsc.store_scatter(sub_buf, [dst], packed, mask=m)
          return off2 + plsc.all_reduce_population_count(m)

        k2 = jnp.max(lax.fori_loop(0, nvec, p2_body, zero16))

        def late_drain_body(j, carry, bn=bn):
          wait_one_write(bn)
          return carry

        lax.fori_loop(0, jnp.minimum(kprev[bn], NOUT), late_drain_body, 0)
        kprev = tuple(jnp.int32(0) if i == bn else kprev[i]
                      for i in range(NRB))

        @pl.when(s + 1 < NS)
        def _(s=s, bn=bn):
          start_load(s + 1, bn)

        wait_load(b)

        def wr_body(jv, carry, b=b, k2=k2):
          v = sub_buf[pl.ds(jv * 16, 16)]
          for l in range(16):
            j = jv * 16 + l
            valid = j < k2

            @pl.when(valid)
            def _(l=l, b=b):
              w = v[l]
              pltpu.async_copy(rowbufs[b].at[pl.ds(w & 7, 1)],
                               out_hbm.at[pl.ds(w >> 3, 1)], osems[b])

            @pl.when(valid & (j >= NOUT))
            def _(b=b):
              wait_one_write(b)

          return carry

        lax.fori_loop(0, (k2 + 15) // 16, wr_body, 0)
        kprev = tuple(k2 if i == b else kprev[i] for i in range(NRB))
      return kprev

    kfin = lax.fori_loop(0, NS // NRB, s_pair_body,
                         tuple(jnp.int32(0) for _ in range(NRB)))

    for b in range(NRB):

      def fin_drain_body(j, carry, b=b):
        wait_one_write(b)
        return carry

      lax.fori_loop(0, jnp.minimum(kfin[b], NOUT), fin_drain_body, 0)

  return kern(table, indices)


def kernel(position_ids, embedding_table):
  idx = jnp.transpose(position_ids).reshape(ROWS).astype(jnp.int32)
  out = _sc_scatter_gather(embedding_table, idx)
  return out.reshape(BATCH, SEQ, HIDDEN)
